# baseline (speedup 1.0000x reference)
import functools

import jax
import jax.numpy as jnp
from jax import lax
from jax.experimental import pallas as pl
from jax.experimental.pallas import tpu as pltpu
from jax.experimental.pallas import tpu_sc as plsc


def _build_sc_kernel(S, V, D, B, num_cores, num_subcores):
    NW = num_cores * num_subcores
    s_per_w = S // NW
    CHS = 4
    CH = CHS * B
    NB = 6
    NCH = s_per_w // CHS
    LG = D // 16

    mesh = plsc.VectorSubcoreMesh(core_axis_name="c", subcore_axis_name="s")

    @functools.partial(
        pl.kernel,
        mesh=mesh,
        out_type=jax.ShapeDtypeStruct((S, B, D), jnp.float32),
        scratch_types=[
            pltpu.VMEM((s_per_w * B,), jnp.int32),
            pltpu.VMEM((B, D), jnp.float32),
        ]
        + [pltpu.VMEM((CH, D), jnp.float32)] * NB
        + [pltpu.SemaphoreType.DMA] * (2 * NB),
    )
    def k(tok_hbm, table_hbm, bias_hbm, out_hbm, idx_v, bias_v, *bufs):
        rows = bufs[:NB]
        gsem = bufs[NB:2 * NB]
        ssem = bufs[2 * NB:]
        wid = lax.axis_index("s") * num_cores + lax.axis_index("c")
        seq_base = wid * s_per_w
        pltpu.sync_copy(tok_hbm.at[pl.ds(seq_base * B, s_per_w * B)], idx_v)
        pltpu.sync_copy(bias_hbm, bias_v)

        def gather(c, b):
            return pltpu.make_async_copy(
                table_hbm.at[idx_v.at[pl.ds(c * CH, CH)]], rows[b], gsem[b]
            )

        class scatter:
            def __init__(self, c, b):
                s0 = seq_base + c * CHS
                self.copies = [
                    pltpu.make_async_copy(
                        rows[b].at[pl.ds(q * B, B)], out_hbm.at[s0 + q], ssem[b]
                    )
                    for q in range(CHS)
                ]

            def start(self):
                for cp in self.copies:
                    cp.start()

            def wait(self):
                for cp in self.copies:
                    cp.wait()

        def add_bias(b):
            rv = rows[b]

            def d_body(d2, dcarry):
                for u in range(2):
                    off = pl.multiple_of((2 * d2 + u) * 16, 16)
                    bvec = [bias_v[jj, pl.ds(off, 16)] for jj in range(B)]
                    for j in range(CH):
                        plsc.addupdate(rv.at[j, pl.ds(off, 16)], bvec[j % B])
                return dcarry

            lax.fori_loop(0, LG // 2, d_body, 0)

        M = NCH // NB
        REM = NCH - M * NB

        for c in range(NB):
            gather(c, c).start()

        def step(i, b, last_group):
            cc = NB * i + b
            gather(cc, b).wait()

            pb = (b - 1) % NB

            def refill():
                scatter(cc - 1, pb).wait()
                gather(cc + NB - 1, pb).start()

            refill_ok = (NB * last_group + b + NB - 1) < NCH
            if b == 0:
                pl.when(i >= 1)(refill)
            elif refill_ok:
                refill()
            else:
                pl.when(i < last_group)(refill)

            add_bias(b)
            scatter(cc, b).start()

        def ring_body(i, carry):
            for b in range(NB):
                step(i, b, M - 1)
            return carry

        lax.fori_loop(0, M, ring_body, 0)
        for r in range(REM):
            cc = M * NB + r
            gather(cc, cc % NB).wait()
            add_bias(cc % NB)
            scatter(cc, cc % NB).start()
            if cc + NB - 1 < NCH:
                scatter(cc - 1, (cc - 1) % NB).wait()
                gather(cc + NB - 1, (cc - 1) % NB).start()
        for c in range(NCH - NB, NCH):
            scatter(c, c % NB).wait()

    return k


def kernel(src_tokens, embed_table, condition_bias):
    S, B = src_tokens.shape
    V, D = embed_table.shape
    tok = src_tokens.reshape(S * B).astype(jnp.int32)
    info = plsc.get_sparse_core_info()
    k = _build_sc_kernel(S, V, D, B, info.num_cores, info.num_subcores)
    return k(tok, embed_table, condition_bias)


# speedup vs baseline: 1.0170x; 1.0170x over previous
import functools

import jax
import jax.numpy as jnp
from jax import lax
from jax.experimental import pallas as pl
from jax.experimental.pallas import tpu as pltpu
from jax.experimental.pallas import tpu_sc as plsc


def _build_sc_kernel(S, V, D, B, num_cores, num_subcores):
    NW = num_cores * num_subcores
    s_per_w = S // NW
    CHS = 8
    CH = CHS * B
    NB = 3
    NCH = s_per_w // CHS
    LG = D // 16

    mesh = plsc.VectorSubcoreMesh(core_axis_name="c", subcore_axis_name="s")

    @functools.partial(
        pl.kernel,
        mesh=mesh,
        out_type=jax.ShapeDtypeStruct((S, B, D), jnp.float32),
        scratch_types=[
            pltpu.VMEM((s_per_w * B,), jnp.int32),
            pltpu.VMEM((B, D), jnp.float32),
        ]
        + [pltpu.VMEM((CH, D), jnp.float32)] * NB
        + [pltpu.SemaphoreType.DMA] * (2 * NB),
    )
    def k(tok_hbm, table_hbm, bias_hbm, out_hbm, idx_v, bias_v, *bufs):
        rows = bufs[:NB]
        gsem = bufs[NB:2 * NB]
        ssem = bufs[2 * NB:]
        wid = lax.axis_index("s") * num_cores + lax.axis_index("c")
        seq_base = wid * s_per_w
        pltpu.sync_copy(tok_hbm.at[pl.ds(seq_base * B, s_per_w * B)], idx_v)
        pltpu.sync_copy(bias_hbm, bias_v)

        def gather(c, b):
            return pltpu.make_async_copy(
                table_hbm.at[idx_v.at[pl.ds(c * CH, CH)]], rows[b], gsem[b]
            )

        class scatter:
            def __init__(self, c, b):
                s0 = seq_base + c * CHS
                self.copies = [
                    pltpu.make_async_copy(
                        rows[b].at[pl.ds(q * B, B)], out_hbm.at[s0 + q], ssem[b]
                    )
                    for q in range(CHS)
                ]

            def start(self):
                for cp in self.copies:
                    cp.start()

            def wait(self):
                for cp in self.copies:
                    cp.wait()

        def add_bias(b):
            rv = rows[b]

            def d_body(d2, dcarry):
                for u in range(2):
                    off = pl.multiple_of((2 * d2 + u) * 16, 16)
                    bvec = [bias_v[jj, pl.ds(off, 16)] for jj in range(B)]
                    for j in range(CH):
                        plsc.addupdate(rv.at[j, pl.ds(off, 16)], bvec[j % B])
                return dcarry

            lax.fori_loop(0, LG // 2, d_body, 0)

        M = NCH // NB
        REM = NCH - M * NB

        for c in range(NB):
            gather(c, c).start()

        def step(i, b, last_group):
            cc = NB * i + b
            gather(cc, b).wait()

            pb = (b - 1) % NB

            def refill():
                scatter(cc - 1, pb).wait()
                gather(cc + NB - 1, pb).start()

            refill_ok = (NB * last_group + b + NB - 1) < NCH
            if b == 0:
                pl.when(i >= 1)(refill)
            elif refill_ok:
                refill()
            else:
                pl.when(i < last_group)(refill)

            add_bias(b)
            scatter(cc, b).start()

        def ring_body(i, carry):
            for b in range(NB):
                step(i, b, M - 1)
            return carry

        lax.fori_loop(0, M, ring_body, 0)
        for r in range(REM):
            cc = M * NB + r
            gather(cc, cc % NB).wait()
            add_bias(cc % NB)
            scatter(cc, cc % NB).start()
            if cc + NB - 1 < NCH:
                scatter(cc - 1, (cc - 1) % NB).wait()
                gather(cc + NB - 1, (cc - 1) % NB).start()
        for c in range(NCH - NB, NCH):
            scatter(c, c % NB).wait()

    return k


def kernel(src_tokens, embed_table, condition_bias):
    S, B = src_tokens.shape
    V, D = embed_table.shape
    tok = src_tokens.reshape(S * B).astype(jnp.int32)
    info = plsc.get_sparse_core_info()
    k = _build_sc_kernel(S, V, D, B, info.num_cores, info.num_subcores)
    return k(tok, embed_table, condition_bias)


# speedup vs baseline: 1.0215x; 1.0044x over previous
import functools

import jax
import jax.numpy as jnp
from jax import lax
from jax.experimental import pallas as pl
from jax.experimental.pallas import tpu as pltpu
from jax.experimental.pallas import tpu_sc as plsc


def _build_sc_kernel(S, V, D, B, num_cores, num_subcores):
    NW = num_cores * num_subcores
    s_per_w = S // NW
    CHS = 8
    CH = CHS * B
    NB = 3
    NCH = s_per_w // CHS
    LG = D // 16

    mesh = plsc.VectorSubcoreMesh(core_axis_name="c", subcore_axis_name="s")

    @functools.partial(
        pl.kernel,
        mesh=mesh,
        out_type=jax.ShapeDtypeStruct((S, B, D), jnp.float32),
        scratch_types=[
            pltpu.VMEM((s_per_w * B,), jnp.int32),
            pltpu.VMEM((B, D), jnp.float32),
        ]
        + [pltpu.VMEM((CH, D), jnp.float32)] * NB
        + [pltpu.SemaphoreType.DMA] * (2 * NB),
    )
    def k(tok_hbm, table_hbm, bias_hbm, out_hbm, idx_v, bias_v, *bufs):
        rows = bufs[:NB]
        gsem = bufs[NB:2 * NB]
        ssem = bufs[2 * NB:]
        wid = lax.axis_index("s") * num_cores + lax.axis_index("c")
        seq_base = wid * s_per_w
        pltpu.sync_copy(tok_hbm.at[pl.ds(seq_base * B, s_per_w * B)], idx_v)
        pltpu.sync_copy(bias_hbm, bias_v)

        def gather(c, b):
            return pltpu.make_async_copy(
                table_hbm.at[idx_v.at[pl.ds(c * CH, CH)]], rows[b], gsem[b]
            )

        class scatter:
            def __init__(self, c, b):
                s0 = seq_base + c * CHS
                self.copies = [
                    pltpu.make_async_copy(
                        rows[b],
                        out_hbm.at[pl.ds(s0, CHS)].reshape(CH, D),
                        ssem[b],
                    )
                ]

            def start(self):
                for cp in self.copies:
                    cp.start()

            def wait(self):
                for cp in self.copies:
                    cp.wait()

        def add_bias(b):
            rv = rows[b]

            def d_body(d2, dcarry):
                for u in range(2):
                    off = pl.multiple_of((2 * d2 + u) * 16, 16)
                    bvec = [bias_v[jj, pl.ds(off, 16)] for jj in range(B)]
                    for j in range(CH):
                        plsc.addupdate(rv.at[j, pl.ds(off, 16)], bvec[j % B])
                return dcarry

            lax.fori_loop(0, LG // 2, d_body, 0)

        M = NCH // NB
        REM = NCH - M * NB

        for c in range(NB):
            gather(c, c).start()

        def step(i, b, last_group):
            cc = NB * i + b
            gather(cc, b).wait()

            pb = (b - 1) % NB

            def refill():
                scatter(cc - 1, pb).wait()
                gather(cc + NB - 1, pb).start()

            refill_ok = (NB * last_group + b + NB - 1) < NCH
            if b == 0:
                pl.when(i >= 1)(refill)
            elif refill_ok:
                refill()
            else:
                pl.when(i < last_group)(refill)

            add_bias(b)
            scatter(cc, b).start()

        def ring_body(i, carry):
            for b in range(NB):
                step(i, b, M - 1)
            return carry

        lax.fori_loop(0, M, ring_body, 0)
        for r in range(REM):
            cc = M * NB + r
            gather(cc, cc % NB).wait()
            add_bias(cc % NB)
            scatter(cc, cc % NB).start()
            if cc + NB - 1 < NCH:
                scatter(cc - 1, (cc - 1) % NB).wait()
                gather(cc + NB - 1, (cc - 1) % NB).start()
        for c in range(NCH - NB, NCH):
            scatter(c, c % NB).wait()

    return k


def kernel(src_tokens, embed_table, condition_bias):
    S, B = src_tokens.shape
    V, D = embed_table.shape
    tok = src_tokens.reshape(S * B).astype(jnp.int32)
    info = plsc.get_sparse_core_info()
    k = _build_sc_kernel(S, V, D, B, info.num_cores, info.num_subcores)
    return k(tok, embed_table, condition_bias)
